# baseline (device time: 28304 ns/iter reference)
import jax
import jax.numpy as jnp
from jax import lax
from jax.experimental import pallas as pl
from jax.experimental.pallas import tpu as pltpu

N_DEV = 16
N_IDX = 1024
V_PER = 4096
D = 512
CH = N_IDX // N_DEV
C = 128
P = 16


def kernel(table, idx):
    assert table.shape == (V_PER, D), table.shape
    assert idx.shape == (N_IDX,), idx.shape
    idx_row = idx.reshape(1, N_IDX)
    idx_col = idx.reshape(N_IDX, 1)

    def body(
        table_ref,
        idx_row_ref,
        idx_col_ref,
        out_ref,
        packed_ref,
        land2_ref,
        s1_send,
        s1_recv,
        s2_send,
        s2_recv,
    ):
        my = lax.axis_index("i")
        tbl16 = table_ref[:, :].astype(jnp.bfloat16)

        local = idx_row_ref[:, :] - my * V_PER
        owned = jnp.logical_and(local >= 0, local < V_PER)
        iota_m = lax.broadcasted_iota(jnp.int32, (N_IDX, N_IDX), 0)
        iota_n = lax.broadcasted_iota(jnp.int32, (N_IDX, N_IDX), 1)
        tri = (iota_m < iota_n).astype(jnp.bfloat16)
        rank = jnp.dot(
            owned.astype(jnp.bfloat16), tri,
            preferred_element_type=jnp.float32,
        ).astype(jnp.int32)
        rank_iota = lax.broadcasted_iota(jnp.int32, (C, N_IDX), 0)
        perm = jnp.logical_and(rank_iota == rank, owned)
        perm32 = perm.astype(jnp.float32)
        local_col = (idx_col_ref[:, :] - my * V_PER).astype(jnp.float32)
        n_col = lax.broadcasted_iota(jnp.int32, (N_IDX, 1), 0).astype(
            jnp.float32
        )
        lvals = jnp.dot(
            perm32, local_col, preferred_element_type=jnp.float32
        ).astype(jnp.int32)
        n_of_r = jnp.dot(
            perm32, n_col, preferred_element_type=jnp.float32
        ).astype(jnp.int32)
        valid = jnp.dot(
            perm32,
            jnp.ones((N_IDX, 1), jnp.float32),
            preferred_element_type=jnp.float32,
        ) > 0
        sel = (
            lvals == lax.broadcasted_iota(jnp.int32, (C, V_PER), 1)
        ).astype(jnp.bfloat16)
        compact = jnp.dot(
            sel, tbl16, preferred_element_type=jnp.float32
        ).astype(jnp.bfloat16)

        t_col = n_of_r // CH
        validf = valid.astype(jnp.bfloat16)
        keyoh = (
            t_col == lax.broadcasted_iota(jnp.int32, (C, N_DEV), 1)
        ).astype(jnp.bfloat16) * validf
        pairs = lax.dot_general(
            keyoh, keyoh,
            dimension_numbers=(((1,), (1,)), ((), ())),
            preferred_element_type=jnp.float32,
        )
        tri_c = (
            lax.broadcasted_iota(jnp.int32, (C, C), 1)
            < lax.broadcasted_iota(jnp.int32, (C, C), 0)
        ).astype(jnp.float32)
        p_col = jnp.sum(pairs * tri_c, axis=1, keepdims=True).astype(
            jnp.int32
        )
        q_tgt = t_col * P + p_col
        pk_ok = jnp.logical_and(valid, p_col < P).astype(jnp.bfloat16)
        pkt = (
            q_tgt == lax.broadcasted_iota(jnp.int32, (C, N_DEV * P), 1)
        ).astype(jnp.bfloat16) * pk_ok
        packed_ref[:, :] = lax.dot_general(
            pkt, compact,
            dimension_numbers=(((0,), (0,)), ((), ())),
            preferred_element_type=jnp.float32,
        ).astype(jnp.bfloat16)

        barrier_sem = pltpu.get_barrier_semaphore()
        for k in range(1, N_DEV):
            peer = lax.rem(my + k, N_DEV)
            pl.semaphore_signal(
                barrier_sem, inc=1,
                device_id=(peer,), device_id_type=pl.DeviceIdType.MESH,
            )
        pl.semaphore_wait(barrier_sem, N_DEV - 1)

        phase1 = []
        for t in range(N_DEV):
            rdma = pltpu.make_async_remote_copy(
                src_ref=packed_ref.at[pl.ds(t * P, P), :],
                dst_ref=land2_ref.at[my],
                send_sem=s1_send.at[t],
                recv_sem=s1_recv,
                device_id=(t,),
                device_id_type=pl.DeviceIdType.MESH,
            )
            rdma.start()
            phase1.append(rdma)

        gidx_c = idx_col_ref[pl.ds(my * CH, CH), :]
        owner_c = gidx_c // V_PER
        ownoh = (
            owner_c == lax.broadcasted_iota(jnp.int32, (CH, N_DEV), 1)
        ).astype(jnp.bfloat16)
        pairs2 = lax.dot_general(
            ownoh, ownoh,
            dimension_numbers=(((1,), (1,)), ((), ())),
            preferred_element_type=jnp.float32,
        )
        tri_j = (
            lax.broadcasted_iota(jnp.int32, (CH, CH), 1)
            < lax.broadcasted_iota(jnp.int32, (CH, CH), 0)
        ).astype(jnp.float32)
        p_j = jnp.sum(pairs2 * tri_j, axis=1, keepdims=True).astype(
            jnp.int32
        )
        q_src = owner_c * P + p_j
        sm = (
            q_src == lax.broadcasted_iota(jnp.int32, (CH, N_DEV * P), 1)
        ).astype(jnp.bfloat16) * (p_j < P).astype(jnp.bfloat16)

        for rdma in phase1:
            rdma.wait_recv()

        blocks = land2_ref[:, :, :].reshape(N_DEV * P, D)
        out_ref[pl.ds(my * CH, CH), :] = jnp.dot(
            sm, blocks, preferred_element_type=jnp.float32
        ).astype(jnp.bfloat16)

        phase2 = []
        for t in range(N_DEV):
            rdma = pltpu.make_async_remote_copy(
                src_ref=out_ref.at[pl.ds(my * CH, CH), :],
                dst_ref=out_ref.at[pl.ds(my * CH, CH), :],
                send_sem=s2_send.at[t],
                recv_sem=s2_recv,
                device_id=(t,),
                device_id_type=pl.DeviceIdType.MESH,
            )
            rdma.start()
            phase2.append(rdma)
        for rdma in phase2:
            rdma.wait_recv()
        for rdma in phase1:
            rdma.wait_send()
        for rdma in phase2:
            rdma.wait_send()

    return pl.pallas_call(
        body,
        out_shape=jax.ShapeDtypeStruct((N_IDX, D), jnp.bfloat16),
        in_specs=[
            pl.BlockSpec(memory_space=pltpu.VMEM),
            pl.BlockSpec(memory_space=pltpu.VMEM),
            pl.BlockSpec(memory_space=pltpu.VMEM),
        ],
        out_specs=pl.BlockSpec(memory_space=pltpu.VMEM),
        scratch_shapes=[
            pltpu.VMEM((N_DEV * P, D), jnp.bfloat16),
            pltpu.VMEM((N_DEV, P, D), jnp.bfloat16),
            pltpu.SemaphoreType.DMA((N_DEV,)),
            pltpu.SemaphoreType.DMA,
            pltpu.SemaphoreType.DMA((N_DEV,)),
            pltpu.SemaphoreType.DMA,
        ],
        compiler_params=pltpu.CompilerParams(collective_id=0),
    )(table, idx_row, idx_col)


# device time: 28077 ns/iter; 1.0081x vs baseline; 1.0081x over previous
import jax
import jax.numpy as jnp
from jax import lax
from jax.experimental import pallas as pl
from jax.experimental.pallas import tpu as pltpu

N_DEV = 16
N_IDX = 1024
V_PER = 4096
D = 512
CH = N_IDX // N_DEV
C = 128
P = 16


def kernel(table, idx):
    assert table.shape == (V_PER, D), table.shape
    assert idx.shape == (N_IDX,), idx.shape
    idx_row = idx.reshape(1, N_IDX)
    idx_col = idx.reshape(N_IDX, 1)

    def body(
        table_ref,
        idx_row_ref,
        idx_col_ref,
        out_ref,
        packed_ref,
        land2_ref,
        s1_send,
        s1_recv,
        s2_send,
        s2_recv,
    ):
        my = lax.axis_index("i")
        tbl16 = table_ref[:, :].astype(jnp.bfloat16)

        local = idx_row_ref[:, :] - my * V_PER
        owned = jnp.logical_and(local >= 0, local < V_PER)
        iota_m = lax.broadcasted_iota(jnp.int32, (N_IDX, N_IDX), 0)
        iota_n = lax.broadcasted_iota(jnp.int32, (N_IDX, N_IDX), 1)
        tri = (iota_m < iota_n).astype(jnp.bfloat16)
        rank = jnp.dot(
            owned.astype(jnp.bfloat16), tri,
            preferred_element_type=jnp.float32,
        ).astype(jnp.int32)
        rank_iota = lax.broadcasted_iota(jnp.int32, (C, N_IDX), 0)
        perm = jnp.logical_and(rank_iota == rank, owned)
        n_iota = lax.broadcasted_iota(jnp.int32, (C, N_IDX), 1)
        lvals = jnp.sum(jnp.where(perm, local, 0), axis=1, keepdims=True)
        n_of_r = jnp.sum(jnp.where(perm, n_iota, 0), axis=1, keepdims=True)
        valid = jnp.sum(
            perm.astype(jnp.int32), axis=1, keepdims=True
        ) > 0
        sel = (
            lvals == lax.broadcasted_iota(jnp.int32, (C, V_PER), 1)
        ).astype(jnp.bfloat16)
        compact = jnp.dot(
            sel, tbl16, preferred_element_type=jnp.float32
        ).astype(jnp.bfloat16)

        t_col = n_of_r // CH
        validf = valid.astype(jnp.bfloat16)
        keyoh = (
            t_col == lax.broadcasted_iota(jnp.int32, (C, N_DEV), 1)
        ).astype(jnp.bfloat16) * validf
        pairs = lax.dot_general(
            keyoh, keyoh,
            dimension_numbers=(((1,), (1,)), ((), ())),
            preferred_element_type=jnp.float32,
        )
        tri_c = (
            lax.broadcasted_iota(jnp.int32, (C, C), 1)
            < lax.broadcasted_iota(jnp.int32, (C, C), 0)
        ).astype(jnp.float32)
        p_col = jnp.sum(pairs * tri_c, axis=1, keepdims=True).astype(
            jnp.int32
        )
        q_tgt = t_col * P + p_col
        pk_ok = jnp.logical_and(valid, p_col < P).astype(jnp.bfloat16)
        pkt = (
            q_tgt == lax.broadcasted_iota(jnp.int32, (C, N_DEV * P), 1)
        ).astype(jnp.bfloat16) * pk_ok
        packed_ref[:, :] = lax.dot_general(
            pkt, compact,
            dimension_numbers=(((0,), (0,)), ((), ())),
            preferred_element_type=jnp.float32,
        ).astype(jnp.bfloat16)

        barrier_sem = pltpu.get_barrier_semaphore()
        for k in range(1, N_DEV):
            peer = lax.rem(my + k, N_DEV)
            pl.semaphore_signal(
                barrier_sem, inc=1,
                device_id=(peer,), device_id_type=pl.DeviceIdType.MESH,
            )
        pl.semaphore_wait(barrier_sem, N_DEV - 1)

        phase1 = []
        for t in range(N_DEV):
            rdma = pltpu.make_async_remote_copy(
                src_ref=packed_ref.at[pl.ds(t * P, P), :],
                dst_ref=land2_ref.at[my],
                send_sem=s1_send.at[t],
                recv_sem=s1_recv,
                device_id=(t,),
                device_id_type=pl.DeviceIdType.MESH,
            )
            rdma.start()
            phase1.append(rdma)

        gidx_c = idx_col_ref[pl.ds(my * CH, CH), :]
        owner_c = gidx_c // V_PER
        ownoh = (
            owner_c == lax.broadcasted_iota(jnp.int32, (CH, N_DEV), 1)
        ).astype(jnp.bfloat16)
        pairs2 = lax.dot_general(
            ownoh, ownoh,
            dimension_numbers=(((1,), (1,)), ((), ())),
            preferred_element_type=jnp.float32,
        )
        tri_j = (
            lax.broadcasted_iota(jnp.int32, (CH, CH), 1)
            < lax.broadcasted_iota(jnp.int32, (CH, CH), 0)
        ).astype(jnp.float32)
        p_j = jnp.sum(pairs2 * tri_j, axis=1, keepdims=True).astype(
            jnp.int32
        )
        q_src = owner_c * P + p_j
        sm = (
            q_src == lax.broadcasted_iota(jnp.int32, (CH, N_DEV * P), 1)
        ).astype(jnp.bfloat16) * (p_j < P).astype(jnp.bfloat16)

        for rdma in phase1:
            rdma.wait_recv()

        blocks = land2_ref[:, :, :].reshape(N_DEV * P, D)
        out_ref[pl.ds(my * CH, CH), :] = jnp.dot(
            sm, blocks, preferred_element_type=jnp.float32
        ).astype(jnp.bfloat16)

        phase2 = []
        for t in range(N_DEV):
            rdma = pltpu.make_async_remote_copy(
                src_ref=out_ref.at[pl.ds(my * CH, CH), :],
                dst_ref=out_ref.at[pl.ds(my * CH, CH), :],
                send_sem=s2_send.at[t],
                recv_sem=s2_recv,
                device_id=(t,),
                device_id_type=pl.DeviceIdType.MESH,
            )
            rdma.start()
            phase2.append(rdma)
        for rdma in phase2:
            rdma.wait_recv()
        for rdma in phase1:
            rdma.wait_send()
        for rdma in phase2:
            rdma.wait_send()

    return pl.pallas_call(
        body,
        out_shape=jax.ShapeDtypeStruct((N_IDX, D), jnp.bfloat16),
        in_specs=[
            pl.BlockSpec(memory_space=pltpu.VMEM),
            pl.BlockSpec(memory_space=pltpu.VMEM),
            pl.BlockSpec(memory_space=pltpu.VMEM),
        ],
        out_specs=pl.BlockSpec(memory_space=pltpu.VMEM),
        scratch_shapes=[
            pltpu.VMEM((N_DEV * P, D), jnp.bfloat16),
            pltpu.VMEM((N_DEV, P, D), jnp.bfloat16),
            pltpu.SemaphoreType.DMA((N_DEV,)),
            pltpu.SemaphoreType.DMA,
            pltpu.SemaphoreType.DMA((N_DEV,)),
            pltpu.SemaphoreType.DMA,
        ],
        compiler_params=pltpu.CompilerParams(collective_id=0),
    )(table, idx_row, idx_col)


# device time: 26333 ns/iter; 1.0748x vs baseline; 1.0662x over previous
import jax
import jax.numpy as jnp
from jax import lax
from jax.experimental import pallas as pl
from jax.experimental.pallas import tpu as pltpu

N_DEV = 16
N_IDX = 1024
V_PER = 4096
D = 512
CH = N_IDX // N_DEV
C = 128
P = 16


def kernel(table, idx):
    assert table.shape == (V_PER, D), table.shape
    assert idx.shape == (N_IDX,), idx.shape
    idx_row = idx.reshape(1, N_IDX)
    idx_col = idx.reshape(N_IDX, 1)

    def body(
        table_ref,
        idx_row_ref,
        idx_col_ref,
        out_ref,
        packed_ref,
        land2_ref,
        s1_send,
        s1_recv,
        s2_send,
        s2_recv,
    ):
        my = lax.axis_index("i")
        tbl16 = table_ref[:, :].astype(jnp.bfloat16)

        local = idx_row_ref[:, :] - my * V_PER
        owned = jnp.logical_and(local >= 0, local < V_PER)
        iota_m = lax.broadcasted_iota(jnp.int32, (N_IDX, N_IDX), 0)
        iota_n = lax.broadcasted_iota(jnp.int32, (N_IDX, N_IDX), 1)
        tri = (iota_m < iota_n).astype(jnp.bfloat16)
        rank = jnp.dot(
            owned.astype(jnp.bfloat16), tri,
            preferred_element_type=jnp.float32,
        ).astype(jnp.int32)
        rank_iota = lax.broadcasted_iota(jnp.int32, (C, N_IDX), 0)
        perm = jnp.logical_and(rank_iota == rank, owned)
        n_iota = lax.broadcasted_iota(jnp.int32, (C, N_IDX), 1)
        lvals = jnp.sum(jnp.where(perm, local, 0), axis=1, keepdims=True)
        n_of_r = jnp.sum(jnp.where(perm, n_iota, 0), axis=1, keepdims=True)
        valid = jnp.sum(
            perm.astype(jnp.int32), axis=1, keepdims=True
        ) > 0
        sel = (
            lvals == lax.broadcasted_iota(jnp.int32, (C, V_PER), 1)
        ).astype(jnp.bfloat16)
        compact = jnp.dot(
            sel, tbl16, preferred_element_type=jnp.float32
        ).astype(jnp.bfloat16)

        t_col = n_of_r // CH
        validf = valid.astype(jnp.bfloat16)
        keyoh = (
            t_col == lax.broadcasted_iota(jnp.int32, (C, N_DEV), 1)
        ).astype(jnp.bfloat16) * validf
        pairs = lax.dot_general(
            keyoh, keyoh,
            dimension_numbers=(((1,), (1,)), ((), ())),
            preferred_element_type=jnp.float32,
        )
        tri_c = (
            lax.broadcasted_iota(jnp.int32, (C, C), 1)
            < lax.broadcasted_iota(jnp.int32, (C, C), 0)
        ).astype(jnp.float32)
        p_col = jnp.sum(pairs * tri_c, axis=1, keepdims=True).astype(
            jnp.int32
        )
        q_tgt = t_col * P + p_col
        pk_ok = jnp.logical_and(valid, p_col < P).astype(jnp.bfloat16)
        pkt = (
            q_tgt == lax.broadcasted_iota(jnp.int32, (C, N_DEV * P), 1)
        ).astype(jnp.bfloat16) * pk_ok
        packed_ref[:, :] = lax.dot_general(
            pkt, compact,
            dimension_numbers=(((0,), (0,)), ((), ())),
            preferred_element_type=jnp.float32,
        ).astype(jnp.bfloat16)

        barrier_sem = pltpu.get_barrier_semaphore()
        for k in range(1, N_DEV):
            peer = lax.rem(my + k, N_DEV)
            pl.semaphore_signal(
                barrier_sem, inc=1,
                device_id=(peer,), device_id_type=pl.DeviceIdType.MESH,
            )
        pl.semaphore_wait(barrier_sem, N_DEV - 1)

        phase1 = []
        for t in range(N_DEV):
            rdma = pltpu.make_async_remote_copy(
                src_ref=packed_ref.at[pl.ds(t * P, P), :],
                dst_ref=land2_ref.at[my],
                send_sem=s1_send.at[t],
                recv_sem=s1_recv,
                device_id=(t,),
                device_id_type=pl.DeviceIdType.MESH,
            )
            rdma.start()
            phase1.append(rdma)

        gidx_c = idx_col_ref[pl.ds(my * CH, CH), :]
        owner_c = gidx_c // V_PER
        ownoh = (
            owner_c == lax.broadcasted_iota(jnp.int32, (CH, N_DEV), 1)
        ).astype(jnp.bfloat16)
        pairs2 = lax.dot_general(
            ownoh, ownoh,
            dimension_numbers=(((1,), (1,)), ((), ())),
            preferred_element_type=jnp.float32,
        )
        tri_j = (
            lax.broadcasted_iota(jnp.int32, (CH, CH), 1)
            < lax.broadcasted_iota(jnp.int32, (CH, CH), 0)
        ).astype(jnp.float32)
        p_j = jnp.sum(pairs2 * tri_j, axis=1, keepdims=True).astype(
            jnp.int32
        )
        q_src = owner_c * P + p_j
        sm = (
            q_src == lax.broadcasted_iota(jnp.int32, (CH, N_DEV * P), 1)
        ).astype(jnp.bfloat16) * (p_j < P).astype(jnp.bfloat16)

        for rdma in phase1:
            rdma.wait_recv()

        blocks = land2_ref[:, :, :].reshape(N_DEV * P, D)
        out_ref[pl.ds(my * CH, CH), :] = jnp.dot(
            sm, blocks, preferred_element_type=jnp.float32
        ).astype(jnp.bfloat16)

        phase2 = []
        for k in range(1, N_DEV):
            tgt = lax.rem(my + k, N_DEV)
            rdma = pltpu.make_async_remote_copy(
                src_ref=out_ref.at[pl.ds(my * CH, CH), :],
                dst_ref=out_ref.at[pl.ds(my * CH, CH), :],
                send_sem=s2_send.at[k - 1],
                recv_sem=s2_recv,
                device_id=(tgt,),
                device_id_type=pl.DeviceIdType.MESH,
            )
            rdma.start()
            phase2.append(rdma)
        for rdma in phase2:
            rdma.wait_recv()
        for rdma in phase1:
            rdma.wait_send()
        for rdma in phase2:
            rdma.wait_send()

    return pl.pallas_call(
        body,
        out_shape=jax.ShapeDtypeStruct((N_IDX, D), jnp.bfloat16),
        in_specs=[
            pl.BlockSpec(memory_space=pltpu.VMEM),
            pl.BlockSpec(memory_space=pltpu.VMEM),
            pl.BlockSpec(memory_space=pltpu.VMEM),
        ],
        out_specs=pl.BlockSpec(memory_space=pltpu.VMEM),
        scratch_shapes=[
            pltpu.VMEM((N_DEV * P, D), jnp.bfloat16),
            pltpu.VMEM((N_DEV, P, D), jnp.bfloat16),
            pltpu.SemaphoreType.DMA((N_DEV,)),
            pltpu.SemaphoreType.DMA,
            pltpu.SemaphoreType.DMA((N_DEV,)),
            pltpu.SemaphoreType.DMA,
        ],
        compiler_params=pltpu.CompilerParams(collective_id=0),
    )(table, idx_row, idx_col)
